# baseline (speedup 1.0000x reference)
import functools

import jax
import jax.numpy as jnp
from jax import lax
from jax.experimental import pallas as pl
from jax.experimental.pallas import tpu as pltpu
from jax.experimental.pallas import tpu_sc as plsc

I_DIV = 1000
NC = 2
NS = 16
NW = NC * NS
L = 16
CH = 2048
BIG = 2 ** 30


def _stage1(E, nkey):
    per_w = E // NW
    n_chunks = per_w // CH
    steps = CH // L
    mesh = plsc.VectorSubcoreMesh(
        core_axis_name="c", subcore_axis_name="s",
        num_cores=NC, num_subcores=NS)

    @functools.partial(
        pl.kernel,
        out_type=(
            jax.ShapeDtypeStruct((NW, nkey), jnp.float32),
            jax.ShapeDtypeStruct((NW, nkey), jnp.float32),
            jax.ShapeDtypeStruct((NW, nkey), jnp.int32),
        ),
        mesh=mesh,
        compiler_params=pltpu.CompilerParams(needs_layout_passes=False),
        scratch_types=[
            pltpu.VMEM((nkey,), jnp.float32),
            pltpu.VMEM((nkey,), jnp.float32),
            pltpu.VMEM((nkey,), jnp.int32),
            pltpu.VMEM((2, CH), jnp.int32),
            pltpu.VMEM((2, CH), jnp.float32),
            pltpu.VMEM((2, CH), jnp.float32),
            pltpu.VMEM((2, CH), jnp.int32),
            pltpu.VMEM((2, CH), jnp.int32),
            pltpu.SemaphoreType.DMA,
            pltpu.SemaphoreType.DMA,
            pltpu.VMEM((16,), jnp.int32),
            pltpu.VMEM((16,), jnp.float32),
            pltpu.VMEM((16,), jnp.float32),
            pltpu.VMEM((16,), jnp.float32),
            pltpu.VMEM((16,), jnp.float32),
            pltpu.VMEM((16,), jnp.int32),
        ],
    )
    def k(key_hbm, zr_hbm, sod_hbm, ei_hbm, ej_hbm, elm_hbm, wz_hbm, wr_hbm,
          den_out, num_out, enc_out,
          den_b, num_b, enc_b, key_c, z_c, sod_c, ei_c, ej_c, sem0, sem1,
          elm_v, wz_v, wr_v, scw, scz, scm):
        cid = lax.axis_index("c")
        sid = lax.axis_index("s")
        wid = sid * NC + cid
        base = wid * per_w

        pltpu.sync_copy(elm_hbm, elm_v)
        pltpu.sync_copy(wz_hbm, wz_v)
        pltpu.sync_copy(wr_hbm, wr_v)

        iota = lax.iota(jnp.int32, L)
        zero16 = jnp.zeros((L,), jnp.float32)
        big16 = jnp.full((L,), BIG, jnp.int32)

        def init_body(t, carry):
            sl = pl.ds(t * L, L)
            den_b[sl] = zero16
            num_b[sl] = zero16
            enc_b[sl] = big16
            return carry

        lax.fori_loop(0, nkey // L, init_body, 0)

        sems = (sem0, sem1)

        def dma_pairs(ci, b):
            off = base + ci * CH
            return (
                (key_hbm.at[pl.ds(off, CH)], key_c.at[b]),
                (zr_hbm.at[pl.ds(off, CH)], z_c.at[b]),
                (sod_hbm.at[pl.ds(off, CH)], sod_c.at[b]),
                (ei_hbm.at[pl.ds(off, CH)], ei_c.at[b]),
                (ej_hbm.at[pl.ds(off, CH)], ej_c.at[b]),
            )

        def start_chunk(ci, b):
            for src, dst in dma_pairs(ci, b):
                pltpu.async_copy(src, dst, sems[b])

        def wait_chunk(ci, b):
            for src, dst in dma_pairs(ci, b):
                pltpu.make_async_copy(src, dst, sems[b]).wait()

        def process_chunk(ci, b, carry):
            off = base + ci * CH

            def step(j, c2):
                sl = pl.ds(j * L, L)
                kv = key_c[b, sl]
                zij = -z_c[b, sl]
                sv = sod_c[b, sl]
                e16 = ei_c[b, sl] * 4 + ej_c[b, sl]
                eij = plsc.load_gather(elm_v, [e16])
                msk = eij >= 0
                eijc = jnp.where(msk, eij, 0)
                wzv = plsc.load_gather(wz_v, [eijc])
                wrv = plsc.load_gather(wr_v, [eijc])
                w = jnp.exp(-(wzv * zij) - wrv * sv)
                w = jnp.where(msk, w, 0.0)
                wz_ = w * zij
                pos = off + j * L + iota
                enc = jnp.where(msk, pos * 8 + eij, BIG)

                kprev = key_c[b, pl.ds(jnp.maximum(j * L - 1, 0), L)]
                knext = key_c[b, pl.ds(jnp.minimum(j * L + 1, CH - L), L)]
                is_start = (iota == 0) | (kv != kprev)
                is_last = (iota == L - 1) | (kv != knext)
                s = plsc.cummax(jnp.where(is_start, iota, 0))
                cw = plsc.cumsum(w)
                cz = plsc.cumsum(wz_)
                vm = plsc.cummax(jnp.where(msk, iota, -1))
                scw[...] = cw
                scz[...] = cz
                scm[...] = vm
                sm1 = jnp.maximum(s - 1, 0)
                prew = jnp.where(s > 0, plsc.load_gather(scw, [sm1]), 0.0)
                prez = jnp.where(s > 0, plsc.load_gather(scz, [sm1]), 0.0)
                vmp = jnp.where(iota > 0,
                                plsc.load_gather(scm, [jnp.maximum(iota - 1, 0)]),
                                -1)
                run_w = cw - prew
                run_z = cz - prez
                fv = msk & (vmp < s)

                old_d = plsc.load_gather(den_b, [kv], mask=is_last)
                plsc.store_scatter(den_b, [kv], old_d + run_w, mask=is_last)
                old_n = plsc.load_gather(num_b, [kv], mask=is_last)
                plsc.store_scatter(num_b, [kv], old_n + run_z, mask=is_last)
                old_e = plsc.load_gather(enc_b, [kv], mask=fv)
                plsc.store_scatter(enc_b, [kv], jnp.minimum(old_e, enc), mask=fv)
                return c2

            lax.fori_loop(0, steps, step, 0)
            return carry

        start_chunk(0, 0)
        start_chunk(1, 1)

        def pair_body(t, carry):
            for b in range(2):
                ci = 2 * t + b
                wait_chunk(ci, b)
                process_chunk(ci, b, 0)

                @pl.when(t + 1 < n_chunks // 2)
                def _():
                    start_chunk(ci + 2, b)

            return carry

        lax.fori_loop(0, n_chunks // 2, pair_body, 0)

        pltpu.sync_copy(den_b, den_out.at[wid])
        pltpu.sync_copy(num_b, num_out.at[wid])
        pltpu.sync_copy(enc_b, enc_out.at[wid])

    return k


def _stage2_body(den_ref, num_ref, enc_ref, mu_ref, sg_ref, out_ref):
    den = jnp.sum(den_ref[...], axis=0)
    num = jnp.sum(num_ref[...], axis=0)
    enc = jnp.min(enc_ref[...], axis=0)
    valid = enc < BIG
    e = jnp.where(valid, jnp.bitwise_and(enc, 7), 0)
    z = num / jnp.where(valid, den, 1.0)
    muv = jnp.zeros_like(z)
    sgv = jnp.ones_like(z)
    for t in range(8):
        sel = e == t
        muv = jnp.where(sel, mu_ref[0, t], muv)
        sgv = jnp.where(sel, sg_ref[0, t], sgv)
    d = (z - muv) / sgv
    c = jnp.exp(-(d * d))
    c = jnp.where(valid, c, 0.0)
    r, cc = den.shape
    kidx = (lax.broadcasted_iota(jnp.int32, (r, cc), 0) * cc
            + lax.broadcasted_iota(jnp.int32, (r, cc), 1))
    idx = jnp.minimum((kidx // I_DIV) * 8 + e, 127)
    acc = jnp.zeros((8, 128), jnp.float32)
    jt3 = lax.broadcasted_iota(jnp.int32, (8, cc, 128), 2)
    for t in range(r // 8):
        idx_sl = idx[t * 8:(t + 1) * 8, :]
        c_sl = c[t * 8:(t + 1) * 8, :]
        eq = idx_sl[:, :, None] == jt3
        acc = acc + jnp.sum(jnp.where(eq, c_sl[:, :, None], 0.0), axis=1)
    out_ref[...] = jnp.sum(acc, axis=0, keepdims=True)


def kernel(vec, sod, wz, wr, mu, sigma, elm, key_ni, ei, ej, num_bch):
    del num_bch
    E = vec.shape[0]
    n_items = wz.shape[0]
    B = 16
    nkey = -(-(B * I_DIV) // 2048) * 2048

    key32 = key_ni.astype(jnp.int32)
    ei32 = ei.astype(jnp.int32)
    ej32 = ej.astype(jnp.int32)
    zraw = vec[:, 2]
    elm16 = elm.astype(jnp.int32).reshape(-1)
    wz16 = jnp.pad(wz.astype(jnp.float32), (0, 16 - n_items))
    wr16 = jnp.pad(wr.astype(jnp.float32), (0, 16 - n_items))

    den32, num32, enc32 = _stage1(E, nkey)(
        key32, zraw, sod, ei32, ej32, elm16, wz16, wr16)

    r = nkey // 128
    den3 = den32.reshape(NW, r, 128)
    num3 = num32.reshape(NW, r, 128)
    enc3 = enc32.reshape(NW, r, 128)
    mu_pad = jnp.zeros((8, 128), jnp.float32).at[0, :n_items].set(mu)
    sg_pad = jnp.ones((8, 128), jnp.float32).at[0, :n_items].set(sigma)

    out = pl.pallas_call(
        _stage2_body,
        out_shape=jax.ShapeDtypeStruct((1, 128), jnp.float32),
    )(den3, num3, enc3, mu_pad, sg_pad)
    return out.reshape(B, n_items)


# speedup vs baseline: 1.0103x; 1.0103x over previous
import functools

import jax
import jax.numpy as jnp
from jax import lax
from jax.experimental import pallas as pl
from jax.experimental.pallas import tpu as pltpu
from jax.experimental.pallas import tpu_sc as plsc

I_DIV = 1000
NC = 2
NS = 16
NW = NC * NS
L = 16
CH = 2048
BIG = 2 ** 30


def _stage1(E, nkey):
    per_w = E // NW
    n_chunks = per_w // CH
    steps = CH // L
    mesh = plsc.VectorSubcoreMesh(
        core_axis_name="c", subcore_axis_name="s",
        num_cores=NC, num_subcores=NS)

    @functools.partial(
        pl.kernel,
        out_type=(
            jax.ShapeDtypeStruct((NW, nkey), jnp.float32),
            jax.ShapeDtypeStruct((NW, nkey), jnp.float32),
            jax.ShapeDtypeStruct((NW, nkey), jnp.int32),
        ),
        mesh=mesh,
        compiler_params=pltpu.CompilerParams(needs_layout_passes=False),
        scratch_types=[
            pltpu.VMEM((nkey,), jnp.float32),
            pltpu.VMEM((nkey,), jnp.float32),
            pltpu.VMEM((nkey,), jnp.int32),
            pltpu.VMEM((2, CH), jnp.int32),
            pltpu.VMEM((2, CH), jnp.float32),
            pltpu.VMEM((2, CH), jnp.float32),
            pltpu.VMEM((2, CH), jnp.int32),
            pltpu.VMEM((2, CH), jnp.int32),
            pltpu.SemaphoreType.DMA,
            pltpu.SemaphoreType.DMA,
            pltpu.VMEM((16,), jnp.int32),
            pltpu.VMEM((16,), jnp.float32),
            pltpu.VMEM((16,), jnp.float32),
            pltpu.VMEM((16,), jnp.int32),
            pltpu.VMEM((16,), jnp.float32),
            pltpu.VMEM((16,), jnp.float32),
            pltpu.VMEM((16,), jnp.int32),
        ],
    )
    def k(key_hbm, zr_hbm, sod_hbm, ei_hbm, ej_hbm, elm_hbm, wz_hbm, wr_hbm,
          den_out, num_out, enc_out,
          den_b, num_b, enc_b, key_c, z_c, sod_c, ei_c, ej_c, sem0, sem1,
          elm_v, wz_v, wr_v, sk, scw, scz, scm):
        cid = lax.axis_index("c")
        sid = lax.axis_index("s")
        wid = sid * NC + cid
        base = wid * per_w

        pltpu.sync_copy(elm_hbm, elm_v)
        pltpu.sync_copy(wz_hbm, wz_v)
        pltpu.sync_copy(wr_hbm, wr_v)

        iota = lax.iota(jnp.int32, L)
        zero16 = jnp.zeros((L,), jnp.float32)
        big16 = jnp.full((L,), BIG, jnp.int32)

        def init_body(t, carry):
            sl = pl.ds(t * L, L)
            den_b[sl] = zero16
            num_b[sl] = zero16
            enc_b[sl] = big16
            return carry

        lax.fori_loop(0, nkey // L, init_body, 0)

        sems = (sem0, sem1)

        def dma_pairs(ci, b):
            off = base + ci * CH
            return (
                (key_hbm.at[pl.ds(off, CH)], key_c.at[b]),
                (zr_hbm.at[pl.ds(off, CH)], z_c.at[b]),
                (sod_hbm.at[pl.ds(off, CH)], sod_c.at[b]),
                (ei_hbm.at[pl.ds(off, CH)], ei_c.at[b]),
                (ej_hbm.at[pl.ds(off, CH)], ej_c.at[b]),
            )

        def start_chunk(ci, b):
            for src, dst in dma_pairs(ci, b):
                pltpu.async_copy(src, dst, sems[b])

        def wait_chunk(ci, b):
            for src, dst in dma_pairs(ci, b):
                pltpu.make_async_copy(src, dst, sems[b]).wait()

        def process_chunk(ci, b, carry):
            off = base + ci * CH

            def body(j):
                sl = pl.ds(j * L, L)
                kv = key_c[b, sl]
                zij = -z_c[b, sl]
                sv = sod_c[b, sl]
                e16 = ei_c[b, sl] * 4 + ej_c[b, sl]
                eij = plsc.load_gather(elm_v, [e16])
                msk = eij >= 0
                eijc = jnp.where(msk, eij, 0)
                wzv = plsc.load_gather(wz_v, [eijc])
                wrv = plsc.load_gather(wr_v, [eijc])
                w = jnp.exp(-(wzv * zij) - wrv * sv)
                w = jnp.where(msk, w, 0.0)
                wz_ = w * zij
                pos = off + j * L + iota
                enc = jnp.where(msk, pos * 8 + eij, BIG)

                sk[...] = kv
                kprev = plsc.load_gather(sk, [jnp.maximum(iota - 1, 0)])
                knext = plsc.load_gather(sk, [jnp.minimum(iota + 1, L - 1)])
                is_start = (iota == 0) | (kv != kprev)
                is_last = (iota == L - 1) | (kv != knext)
                s = plsc.cummax(jnp.where(is_start, iota, 0))
                cw = plsc.cumsum(w)
                cz = plsc.cumsum(wz_)
                cm = plsc.cumsum(msk.astype(jnp.int32))
                scw[...] = cw
                scz[...] = cz
                scm[...] = cm
                sm1 = jnp.maximum(s - 1, 0)
                prew = jnp.where(s > 0, plsc.load_gather(scw, [sm1]), 0.0)
                prez = jnp.where(s > 0, plsc.load_gather(scz, [sm1]), 0.0)
                prem = jnp.where(s > 0, plsc.load_gather(scm, [sm1]), 0)
                cml = jnp.where(iota > 0,
                                plsc.load_gather(scm, [jnp.maximum(iota - 1, 0)]),
                                0)
                run_w = cw - prew
                run_z = cz - prez
                fv = msk & ((cml - prem) == 0)

                old_d = plsc.load_gather(den_b, [kv], mask=is_last)
                plsc.store_scatter(den_b, [kv], old_d + run_w, mask=is_last)
                old_n = plsc.load_gather(num_b, [kv], mask=is_last)
                plsc.store_scatter(num_b, [kv], old_n + run_z, mask=is_last)
                old_e = plsc.load_gather(enc_b, [kv], mask=fv)
                plsc.store_scatter(enc_b, [kv], jnp.minimum(old_e, enc), mask=fv)

            def step_pair(jj, c2):
                body(2 * jj)
                body(2 * jj + 1)
                return c2

            lax.fori_loop(0, steps // 2, step_pair, 0)
            return carry

        start_chunk(0, 0)
        start_chunk(1, 1)

        def pair_body(t, carry):
            for b in range(2):
                ci = 2 * t + b
                wait_chunk(ci, b)
                process_chunk(ci, b, 0)

                @pl.when(t + 1 < n_chunks // 2)
                def _():
                    start_chunk(ci + 2, b)

            return carry

        lax.fori_loop(0, n_chunks // 2, pair_body, 0)

        pltpu.sync_copy(den_b, den_out.at[wid])
        pltpu.sync_copy(num_b, num_out.at[wid])
        pltpu.sync_copy(enc_b, enc_out.at[wid])

    return k


def _stage2_body(den_ref, num_ref, enc_ref, mu_ref, sg_ref, out_ref):
    den = jnp.sum(den_ref[...], axis=0)
    num = jnp.sum(num_ref[...], axis=0)
    enc = jnp.min(enc_ref[...], axis=0)
    valid = enc < BIG
    e = jnp.where(valid, jnp.bitwise_and(enc, 7), 0)
    z = num / jnp.where(valid, den, 1.0)
    muv = jnp.zeros_like(z)
    sgv = jnp.ones_like(z)
    for t in range(8):
        sel = e == t
        muv = jnp.where(sel, mu_ref[0, t], muv)
        sgv = jnp.where(sel, sg_ref[0, t], sgv)
    d = (z - muv) / sgv
    c = jnp.exp(-(d * d))
    c = jnp.where(valid, c, 0.0)
    r, cc = den.shape
    kidx = (lax.broadcasted_iota(jnp.int32, (r, cc), 0) * cc
            + lax.broadcasted_iota(jnp.int32, (r, cc), 1))
    idx = jnp.minimum((kidx // I_DIV) * 8 + e, 127)
    acc = jnp.zeros((8, 128), jnp.float32)
    jt3 = lax.broadcasted_iota(jnp.int32, (8, cc, 128), 2)
    for t in range(r // 8):
        idx_sl = idx[t * 8:(t + 1) * 8, :]
        c_sl = c[t * 8:(t + 1) * 8, :]
        eq = idx_sl[:, :, None] == jt3
        acc = acc + jnp.sum(jnp.where(eq, c_sl[:, :, None], 0.0), axis=1)
    out_ref[...] = jnp.sum(acc, axis=0, keepdims=True)


def kernel(vec, sod, wz, wr, mu, sigma, elm, key_ni, ei, ej, num_bch):
    del num_bch
    E = vec.shape[0]
    n_items = wz.shape[0]
    B = 16
    nkey = -(-(B * I_DIV) // 2048) * 2048

    key32 = key_ni.astype(jnp.int32)
    ei32 = ei.astype(jnp.int32)
    ej32 = ej.astype(jnp.int32)
    zraw = vec[:, 2]
    elm16 = elm.astype(jnp.int32).reshape(-1)
    wz16 = jnp.pad(wz.astype(jnp.float32), (0, 16 - n_items))
    wr16 = jnp.pad(wr.astype(jnp.float32), (0, 16 - n_items))

    den32, num32, enc32 = _stage1(E, nkey)(
        key32, zraw, sod, ei32, ej32, elm16, wz16, wr16)

    r = nkey // 128
    den3 = den32.reshape(NW, r, 128)
    num3 = num32.reshape(NW, r, 128)
    enc3 = enc32.reshape(NW, r, 128)
    mu_pad = jnp.zeros((8, 128), jnp.float32).at[0, :n_items].set(mu)
    sg_pad = jnp.ones((8, 128), jnp.float32).at[0, :n_items].set(sigma)

    out = pl.pallas_call(
        _stage2_body,
        out_shape=jax.ShapeDtypeStruct((1, 128), jnp.float32),
    )(den3, num3, enc3, mu_pad, sg_pad)
    return out.reshape(B, n_items)
